# baseline (speedup 1.0000x reference)
import functools

import jax
import jax.numpy as jnp
from jax import lax
from jax.experimental import pallas as pl
from jax.experimental.pallas import tpu as pltpu
from jax.experimental.pallas import tpu_sc as plsc

NU, NI, NCAT, D = 50000, 20000, 1000, 64
NS, LANES = 16, 16
CH = 128
NBUF = 3


def _round_up(x, m):
    return (x + m - 1) // m * m


def _make_spmm(n_src, n_dst, e_half):
    n_src_pad = _round_up(n_src, 256)
    n_dst_pad = _round_up(n_dst, 256)
    e_pad = _round_up(e_half, NS * CH * NBUF)
    nchunks = e_pad // (NS * CH)
    r0 = n_src_pad // 2
    r1 = n_dst_pad // 2
    n_pad = n_src_pad + n_dst_pad
    acc_rows = max(r0, r1) + 8
    passes = ((r0, 0, 0), (r1, n_src, n_src_pad))

    mesh = plsc.VectorSubcoreMesh(core_axis_name="c", subcore_axis_name="s")

    @functools.partial(
        pl.kernel,
        out_type=jax.ShapeDtypeStruct((n_pad, D), jnp.float32),
        mesh=mesh,
        compiler_params=pltpu.CompilerParams(use_tc_tiling_on_sc=False),
        scratch_types=(
            [pltpu.VMEM_SHARED((acc_rows, D), jnp.float32)]
            + [pltpu.VMEM((CH,), jnp.int32), pltpu.VMEM((CH,), jnp.int32),
               pltpu.VMEM((CH,), jnp.float32)] * NBUF
            + [pltpu.VMEM((NBUF, CH), jnp.int32)]
            + [pltpu.VMEM((CH, D), jnp.float32)] * NBUF
            + [pltpu.VMEM((32, D), jnp.float32)]
            + [pltpu.SemaphoreType.DMA] * NBUF
        ),
    )
    def spmm(rows_h, cols_h, vals_h, x_h, out_h, acc, *scr):
        stage = [scr[3 * q:3 * q + 3] for q in range(NBUF)]
        sidx = scr[3 * NBUF]
        gbuf = scr[3 * NBUF + 1:4 * NBUF + 1]
        zbuf = scr[4 * NBUF + 1]
        sems = scr[4 * NBUF + 2:]
        c = lax.axis_index("c")
        s = lax.axis_index("s")

        zero16 = jnp.zeros((LANES,), jnp.float32)

        @pl.loop(0, 32)
        def _(i):
            for j in range(D // LANES):
                zbuf[i, pl.ds(j * LANES, LANES)] = zero16

        for p, (r, base0, out0) in enumerate(passes):
            base = base0 + c * r
            out_off = out0 + c * r
            nrt = r // NS
            full, tail = nrt // 32, nrt % 32
            zbase = s * nrt

            @pl.loop(0, full)
            def _(i):
                pltpu.sync_copy(zbuf, acc.at[pl.ds(zbase + i * 32, 32)])

            if tail:
                pltpu.sync_copy(zbuf.at[pl.ds(0, tail)],
                                acc.at[pl.ds(zbase + full * 32, tail)])
            plsc.subcore_barrier()

            ebase0 = p * e_pad + s * (nchunks * CH)

            def _sidx(rows_v, slot):
                for i in range(CH // LANES):
                    rr = rows_v[pl.ds(i * LANES, LANES)]
                    loc = rr - base
                    inb = (loc >= 0) & (loc < r)
                    idx = jnp.where(inb, loc, r)
                    sidx[slot, pl.ds(i * LANES, LANES)] = idx

            def _scale(vals_v, gb):
                @pl.loop(0, CH // LANES)
                def _(g):
                    vv = vals_v[pl.ds(g * LANES, LANES)]
                    for j in range(LANES):
                        e = g * LANES + j
                        spl = jnp.broadcast_to(
                            lax.slice(vv, (j,), (j + 1,)), (LANES,))
                        for kk in range(D // LANES):
                            gb[e, pl.ds(kk * LANES, LANES)] = (
                                gb[e, pl.ds(kk * LANES, LANES)] * spl)

            @pl.loop(0, nchunks // NBUF)
            def _(t):
                eb0 = ebase0 + t * (NBUF * CH)
                sts = []
                for q in range(NBUF):
                    rv, cv, vv = stage[q]
                    eb = eb0 + q * CH
                    sts.append([
                        pltpu.async_copy(rows_h.at[pl.ds(eb, CH)], rv, sems[q]),
                        pltpu.async_copy(cols_h.at[pl.ds(eb, CH)], cv, sems[q]),
                        pltpu.async_copy(vals_h.at[pl.ds(eb, CH)], vv, sems[q]),
                    ])
                gds = []
                for q in range(NBUF):
                    for x in sts[q]:
                        x.wait()
                    _sidx(stage[q][0], q)
                    gds.append(pltpu.async_copy(x_h.at[stage[q][1]],
                                                gbuf[q], sems[q]))
                scs = []
                for q in range(NBUF):
                    gds[q].wait()
                    _scale(stage[q][2], gbuf[q])
                    scs.append(pltpu.async_copy(gbuf[q], acc.at[sidx.at[q]],
                                                sems[q], add=True))
                for q in range(NBUF):
                    scs[q].wait()

            plsc.subcore_barrier()
            pltpu.sync_copy(acc.at[pl.ds(s * nrt, nrt)],
                            out_h.at[pl.ds(out_off + s * nrt, nrt)])
            plsc.subcore_barrier()

    return spmm, n_src_pad, n_dst_pad, e_pad


_SPMM_UI = _make_spmm(NU, NI, 800000)
_SPMM_UC = _make_spmm(NU, NCAT, 200000)
_SPMM_IC = _make_spmm(NI, NCAT, 40000)
NU_PAD = _SPMM_UI[1]


def _prep_edges(rows, cols, vals, n_src, n_src_pad, e_half, e_pad):
    cols = jnp.where(cols < n_src, cols, cols + (n_src_pad - n_src))
    pad = e_pad - e_half
    z = jnp.zeros((pad,), jnp.int32)
    zf = jnp.zeros((pad,), jnp.float32)
    rows_p = jnp.concatenate([rows[:e_half], z, rows[e_half:], z])
    cols_p = jnp.concatenate([cols[:e_half], z, cols[e_half:], z])
    vals_p = jnp.concatenate([vals[:e_half], zf, vals[e_half:], zf])
    return rows_p, cols_p, vals_p


def _pad_ego(src_emb, dst_emb, n_src_pad, n_dst_pad):
    n_src, n_dst = src_emb.shape[0], dst_emb.shape[0]
    return jnp.concatenate([
        src_emb,
        jnp.zeros((n_src_pad - n_src, D), jnp.float32),
        dst_emb,
        jnp.zeros((n_dst_pad - n_dst, D), jnp.float32),
    ], axis=0)


def _propagate(spmm_pack, rows, cols, vals, src_emb, dst_emb, e_half):
    spmm, n_src_pad, n_dst_pad, e_pad = spmm_pack
    n_src = src_emb.shape[0]
    rows_p, cols_p, vals_p = _prep_edges(rows, cols, vals, n_src,
                                         n_src_pad, e_half, e_pad)
    ego = _pad_ego(src_emb, dst_emb, n_src_pad, n_dst_pad)
    h1 = spmm(rows_p, cols_p, vals_p, ego)
    h2 = spmm(rows_p, cols_p, vals_p, h1)
    return ego, h1, h2


def _make_gather_mean():
    mesh = plsc.VectorSubcoreMesh(core_axis_name="c", subcore_axis_name="s")
    B = 4096
    RPW = B // 32

    @functools.partial(
        pl.kernel,
        out_type=[jax.ShapeDtypeStruct((B, D), jnp.float32)] * 7,
        mesh=mesh,
        compiler_params=pltpu.CompilerParams(use_tc_tiling_on_sc=False),
        scratch_types=[
            pltpu.VMEM((RPW,), jnp.int32),
            pltpu.VMEM((RPW, D), jnp.float32),
            pltpu.VMEM((RPW, D), jnp.float32),
            pltpu.VMEM((RPW, D), jnp.float32),
            pltpu.SemaphoreType.DMA,
        ],
    )
    def gm(ego1, h11, h21, ego2, h12, h22, ego3, h13, h23,
           i0, i1, i2, i3, i4, i5, i6,
           o0, o1, o2, o3, o4, o5, o6,
           idxv, g1, g2, g3, sem):
        c = lax.axis_index("c")
        s = lax.axis_index("s")
        wid = s * 2 + c
        base = wid * RPW
        third = jnp.full((LANES,), 1.0 / 3.0, jnp.float32)
        sections = [
            (ego1, h11, h21, i0, o0), (ego1, h11, h21, i1, o1),
            (ego1, h11, h21, i2, o2), (ego2, h12, h22, i3, o3),
            (ego2, h12, h22, i4, o4), (ego3, h13, h23, i5, o5),
            (ego3, h13, h23, i6, o6),
        ]
        for eg, ha, hb, idx, out in sections:
            pltpu.sync_copy(idx.at[pl.ds(base, RPW)], idxv)
            c1 = pltpu.async_copy(eg.at[idxv], g1, sem)
            c2 = pltpu.async_copy(ha.at[idxv], g2, sem)
            c3 = pltpu.async_copy(hb.at[idxv], g3, sem)
            c1.wait()
            c2.wait()
            c3.wait()

            @pl.loop(0, RPW)
            def _(i):
                for j in range(D // LANES):
                    sl = pl.ds(j * LANES, LANES)
                    g1[i, sl] = (g1[i, sl] + g2[i, sl] + g3[i, sl]) * third

            pltpu.sync_copy(g1, out.at[pl.ds(base, RPW)])

    return gm


_GATHER_MEAN = _make_gather_mean()


def _sup_body(u_ref, p_ref, n_ref, o_ref):
    u = u_ref[...]
    o_ref[...] = (jnp.sum(u * p_ref[...], axis=1, keepdims=True)
                  - jnp.sum(u * n_ref[...], axis=1, keepdims=True))


def _con_body(u_ref, c_ref, o_ref):
    i = pl.program_id(0)
    u = u_ref[...]
    cf = c_ref[...]
    un = u / jnp.maximum(
        jnp.sqrt(jnp.sum(u * u, axis=1, keepdims=True)), 1e-12)
    cn = cf / jnp.maximum(
        jnp.sqrt(jnp.sum(cf * cf, axis=1, keepdims=True)), 1e-12)
    cb = c_ref[pl.ds(i * 256, 256), :]
    cd = cb / jnp.maximum(
        jnp.sqrt(jnp.sum(cb * cb, axis=1, keepdims=True)), 1e-12)
    pos = jnp.sum(un * cd, axis=1)
    o_ref[...] = jax.lax.dot_general(
        un, cn, (((1,), (1,)), ((), ())),
        preferred_element_type=jnp.float32,
        precision=lax.Precision.HIGHEST) - pos[:, None]


def _con_logits(f_u, f_c):
    b = f_u.shape[0]
    return pl.pallas_call(
        _con_body,
        grid=(b // 256,),
        in_specs=[
            pl.BlockSpec((256, D), lambda i: (i, 0)),
            pl.BlockSpec((b, D), lambda i: (0, 0)),
        ],
        out_specs=pl.BlockSpec((256, b), lambda i: (i, 0)),
        out_shape=jax.ShapeDtypeStruct((b, b), jnp.float32),
    )(f_u, f_c)


def kernel(users, pos_items, neg_items, users1, cates1, items2, cates2,
           user_emb, item_emb, cate_emb,
           rows_ui, cols_ui, vals_ui,
           rows_uc, cols_uc, vals_uc,
           rows_ic, cols_ic, vals_ic):
    ego1, h11, h21 = _propagate(_SPMM_UI, rows_ui, cols_ui, vals_ui,
                                user_emb, item_emb, 800000)
    ego2, h12, h22 = _propagate(_SPMM_UC, rows_uc, cols_uc, vals_uc,
                                user_emb, cate_emb, 200000)
    ego3, h13, h23 = _propagate(_SPMM_IC, rows_ic, cols_ic, vals_ic,
                                item_emb, cate_emb, 40000)

    ni_pad_ic = _SPMM_IC[1]
    fs = _GATHER_MEAN(
        ego1, h11, h21, ego2, h12, h22, ego3, h13, h23,
        users, NU_PAD + pos_items, NU_PAD + neg_items,
        users1, NU_PAD + cates1, items2, ni_pad_ic + cates2)
    f_u, f_p, f_n, f_u1, f_c1, f_i2, f_c2 = fs

    sup = pl.pallas_call(
        _sup_body,
        out_shape=jax.ShapeDtypeStruct((f_u.shape[0], 1), jnp.float32),
    )(f_u, f_p, f_n)[:, 0]

    con_u = _con_logits(f_u1, f_c1)
    con_i = _con_logits(f_i2, f_c2)
    return (sup, con_u, con_i)


# speedup vs baseline: 1.0266x; 1.0266x over previous
import functools

import jax
import jax.numpy as jnp
from jax import lax
from jax.experimental import pallas as pl
from jax.experimental.pallas import tpu as pltpu
from jax.experimental.pallas import tpu_sc as plsc

NU, NI, NCAT, D = 50000, 20000, 1000, 64
NS, LANES = 16, 16
CH = 128
NBUF = 2


def _round_up(x, m):
    return (x + m - 1) // m * m


def _make_spmm(n_src, n_dst, e_half):
    n_src_pad = _round_up(n_src, 256)
    n_dst_pad = _round_up(n_dst, 256)
    e_pad = _round_up(e_half, NS * CH * NBUF)
    nchunks = e_pad // (NS * CH)
    r0 = n_src_pad // 2
    r1 = n_dst_pad // 2
    n_pad = n_src_pad + n_dst_pad
    acc_rows = max(r0, r1) + 8
    passes = ((r0, 0, 0), (r1, n_src, n_src_pad))

    mesh = plsc.VectorSubcoreMesh(core_axis_name="c", subcore_axis_name="s")

    @functools.partial(
        pl.kernel,
        out_type=jax.ShapeDtypeStruct((n_pad, D), jnp.float32),
        mesh=mesh,
        compiler_params=pltpu.CompilerParams(use_tc_tiling_on_sc=False),
        scratch_types=(
            [pltpu.VMEM_SHARED((acc_rows, D), jnp.float32)]
            + [pltpu.VMEM((CH,), jnp.int32), pltpu.VMEM((CH,), jnp.int32),
               pltpu.VMEM((CH,), jnp.float32)] * NBUF
            + [pltpu.VMEM((NBUF, CH), jnp.int32)]
            + [pltpu.VMEM((CH, D), jnp.float32)] * NBUF
            + [pltpu.VMEM((32, D), jnp.float32)]
            + [pltpu.SemaphoreType.DMA] * NBUF
        ),
    )
    def spmm(rows_h, cols_h, vals_h, x_h, out_h, acc, *scr):
        stage = [scr[3 * q:3 * q + 3] for q in range(NBUF)]
        sidx = scr[3 * NBUF]
        gbuf = scr[3 * NBUF + 1:4 * NBUF + 1]
        zbuf = scr[4 * NBUF + 1]
        sems = scr[4 * NBUF + 2:]
        c = lax.axis_index("c")
        s = lax.axis_index("s")

        zero16 = jnp.zeros((LANES,), jnp.float32)

        @pl.loop(0, 32)
        def _(i):
            for j in range(D // LANES):
                zbuf[i, pl.ds(j * LANES, LANES)] = zero16

        for p, (r, base0, out0) in enumerate(passes):
            base = base0 + c * r
            out_off = out0 + c * r
            nrt = r // NS
            full, tail = nrt // 32, nrt % 32
            zbase = s * nrt

            @pl.loop(0, full)
            def _(i):
                pltpu.sync_copy(zbuf, acc.at[pl.ds(zbase + i * 32, 32)])

            if tail:
                pltpu.sync_copy(zbuf.at[pl.ds(0, tail)],
                                acc.at[pl.ds(zbase + full * 32, tail)])
            plsc.subcore_barrier()

            ebase0 = p * e_pad + s * (nchunks * CH)

            def _sidx(rows_v, slot):
                for i in range(CH // LANES):
                    rr = rows_v[pl.ds(i * LANES, LANES)]
                    loc = rr - base
                    inb = (loc >= 0) & (loc < r)
                    idx = jnp.where(inb, loc, r)
                    sidx[slot, pl.ds(i * LANES, LANES)] = idx

            def _scale(vals_v, gb):
                @pl.loop(0, CH // LANES)
                def _(g):
                    vv = vals_v[pl.ds(g * LANES, LANES)]
                    for j in range(LANES):
                        e = g * LANES + j
                        spl = jnp.broadcast_to(
                            lax.slice(vv, (j,), (j + 1,)), (LANES,))
                        for kk in range(D // LANES):
                            gb[e, pl.ds(kk * LANES, LANES)] = (
                                gb[e, pl.ds(kk * LANES, LANES)] * spl)

            @pl.loop(0, nchunks // NBUF)
            def _(t):
                eb0 = ebase0 + t * (NBUF * CH)
                sts = []
                for q in range(NBUF):
                    rv, cv, vv = stage[q]
                    eb = eb0 + q * CH
                    sts.append([
                        pltpu.async_copy(rows_h.at[pl.ds(eb, CH)], rv, sems[q]),
                        pltpu.async_copy(cols_h.at[pl.ds(eb, CH)], cv, sems[q]),
                        pltpu.async_copy(vals_h.at[pl.ds(eb, CH)], vv, sems[q]),
                    ])
                gds = []
                for q in range(NBUF):
                    for x in sts[q]:
                        x.wait()
                    _sidx(stage[q][0], q)
                    gds.append(pltpu.async_copy(x_h.at[stage[q][1]],
                                                gbuf[q], sems[q]))
                scs = []
                for q in range(NBUF):
                    gds[q].wait()
                    _scale(stage[q][2], gbuf[q])
                    scs.append(pltpu.async_copy(gbuf[q], acc.at[sidx.at[q]],
                                                sems[q], add=True))
                for q in range(NBUF):
                    scs[q].wait()

            plsc.subcore_barrier()
            pltpu.sync_copy(acc.at[pl.ds(s * nrt, nrt)],
                            out_h.at[pl.ds(out_off + s * nrt, nrt)])
            plsc.subcore_barrier()

    return spmm, n_src_pad, n_dst_pad, e_pad


_SPMM_UI = _make_spmm(NU, NI, 800000)
_SPMM_UC = _make_spmm(NU, NCAT, 200000)
_SPMM_IC = _make_spmm(NI, NCAT, 40000)
NU_PAD = _SPMM_UI[1]


def _prep_edges(rows, cols, vals, n_src, n_src_pad, e_half, e_pad):
    cols = jnp.where(cols < n_src, cols, cols + (n_src_pad - n_src))
    pad = e_pad - e_half
    z = jnp.zeros((pad,), jnp.int32)
    zf = jnp.zeros((pad,), jnp.float32)
    rows_p = jnp.concatenate([rows[:e_half], z, rows[e_half:], z])
    cols_p = jnp.concatenate([cols[:e_half], z, cols[e_half:], z])
    vals_p = jnp.concatenate([vals[:e_half], zf, vals[e_half:], zf])
    return rows_p, cols_p, vals_p


def _pad_ego(src_emb, dst_emb, n_src_pad, n_dst_pad):
    n_src, n_dst = src_emb.shape[0], dst_emb.shape[0]
    return jnp.concatenate([
        src_emb,
        jnp.zeros((n_src_pad - n_src, D), jnp.float32),
        dst_emb,
        jnp.zeros((n_dst_pad - n_dst, D), jnp.float32),
    ], axis=0)


def _propagate(spmm_pack, rows, cols, vals, src_emb, dst_emb, e_half):
    spmm, n_src_pad, n_dst_pad, e_pad = spmm_pack
    n_src = src_emb.shape[0]
    rows_p, cols_p, vals_p = _prep_edges(rows, cols, vals, n_src,
                                         n_src_pad, e_half, e_pad)
    ego = _pad_ego(src_emb, dst_emb, n_src_pad, n_dst_pad)
    h1 = spmm(rows_p, cols_p, vals_p, ego)
    h2 = spmm(rows_p, cols_p, vals_p, h1)
    return ego, h1, h2


def _make_gather_mean():
    mesh = plsc.VectorSubcoreMesh(core_axis_name="c", subcore_axis_name="s")
    B = 4096
    RPW = B // 32

    @functools.partial(
        pl.kernel,
        out_type=[jax.ShapeDtypeStruct((B, D), jnp.float32)] * 7,
        mesh=mesh,
        compiler_params=pltpu.CompilerParams(use_tc_tiling_on_sc=False),
        scratch_types=[
            pltpu.VMEM((RPW,), jnp.int32),
            pltpu.VMEM((RPW, D), jnp.float32),
            pltpu.VMEM((RPW, D), jnp.float32),
            pltpu.VMEM((RPW, D), jnp.float32),
            pltpu.SemaphoreType.DMA,
        ],
    )
    def gm(ego1, h11, h21, ego2, h12, h22, ego3, h13, h23,
           i0, i1, i2, i3, i4, i5, i6,
           o0, o1, o2, o3, o4, o5, o6,
           idxv, g1, g2, g3, sem):
        c = lax.axis_index("c")
        s = lax.axis_index("s")
        wid = s * 2 + c
        base = wid * RPW
        third = jnp.full((LANES,), 1.0 / 3.0, jnp.float32)
        sections = [
            (ego1, h11, h21, i0, o0), (ego1, h11, h21, i1, o1),
            (ego1, h11, h21, i2, o2), (ego2, h12, h22, i3, o3),
            (ego2, h12, h22, i4, o4), (ego3, h13, h23, i5, o5),
            (ego3, h13, h23, i6, o6),
        ]
        for eg, ha, hb, idx, out in sections:
            pltpu.sync_copy(idx.at[pl.ds(base, RPW)], idxv)
            c1 = pltpu.async_copy(eg.at[idxv], g1, sem)
            c2 = pltpu.async_copy(ha.at[idxv], g2, sem)
            c3 = pltpu.async_copy(hb.at[idxv], g3, sem)
            c1.wait()
            c2.wait()
            c3.wait()

            @pl.loop(0, RPW)
            def _(i):
                for j in range(D // LANES):
                    sl = pl.ds(j * LANES, LANES)
                    g1[i, sl] = (g1[i, sl] + g2[i, sl] + g3[i, sl]) * third

            pltpu.sync_copy(g1, out.at[pl.ds(base, RPW)])

    return gm


_GATHER_MEAN = _make_gather_mean()


def _sup_body(u_ref, p_ref, n_ref, o_ref):
    u = u_ref[...]
    o_ref[...] = (jnp.sum(u * p_ref[...], axis=1, keepdims=True)
                  - jnp.sum(u * n_ref[...], axis=1, keepdims=True))


def _con_body(u_ref, c_ref, o_ref):
    i = pl.program_id(0)
    u = u_ref[...]
    cf = c_ref[...]
    un = u / jnp.maximum(
        jnp.sqrt(jnp.sum(u * u, axis=1, keepdims=True)), 1e-12)
    cn = cf / jnp.maximum(
        jnp.sqrt(jnp.sum(cf * cf, axis=1, keepdims=True)), 1e-12)
    cb = c_ref[pl.ds(i * 256, 256), :]
    cd = cb / jnp.maximum(
        jnp.sqrt(jnp.sum(cb * cb, axis=1, keepdims=True)), 1e-12)
    pos = jnp.sum(un * cd, axis=1)
    o_ref[...] = jax.lax.dot_general(
        un, cn, (((1,), (1,)), ((), ())),
        preferred_element_type=jnp.float32,
        precision=lax.Precision.HIGHEST) - pos[:, None]


def _con_logits(f_u, f_c):
    b = f_u.shape[0]
    return pl.pallas_call(
        _con_body,
        grid=(b // 256,),
        in_specs=[
            pl.BlockSpec((256, D), lambda i: (i, 0)),
            pl.BlockSpec((b, D), lambda i: (0, 0)),
        ],
        out_specs=pl.BlockSpec((256, b), lambda i: (i, 0)),
        out_shape=jax.ShapeDtypeStruct((b, b), jnp.float32),
    )(f_u, f_c)


def kernel(users, pos_items, neg_items, users1, cates1, items2, cates2,
           user_emb, item_emb, cate_emb,
           rows_ui, cols_ui, vals_ui,
           rows_uc, cols_uc, vals_uc,
           rows_ic, cols_ic, vals_ic):
    ego1, h11, h21 = _propagate(_SPMM_UI, rows_ui, cols_ui, vals_ui,
                                user_emb, item_emb, 800000)
    ego2, h12, h22 = _propagate(_SPMM_UC, rows_uc, cols_uc, vals_uc,
                                user_emb, cate_emb, 200000)
    ego3, h13, h23 = _propagate(_SPMM_IC, rows_ic, cols_ic, vals_ic,
                                item_emb, cate_emb, 40000)

    ni_pad_ic = _SPMM_IC[1]
    fs = _GATHER_MEAN(
        ego1, h11, h21, ego2, h12, h22, ego3, h13, h23,
        users, NU_PAD + pos_items, NU_PAD + neg_items,
        users1, NU_PAD + cates1, items2, ni_pad_ic + cates2)
    f_u, f_p, f_n, f_u1, f_c1, f_i2, f_c2 = fs

    sup = pl.pallas_call(
        _sup_body,
        out_shape=jax.ShapeDtypeStruct((f_u.shape[0], 1), jnp.float32),
    )(f_u, f_p, f_n)[:, 0]

    con_u = _con_logits(f_u1, f_c1)
    con_i = _con_logits(f_i2, f_c2)
    return (sup, con_u, con_i)
